# Initial kernel scaffold; baseline (speedup 1.0000x reference)
import functools

import jax
import jax.numpy as jnp
from jax import lax
from jax.experimental import pallas as pl
from jax.experimental.pallas import tpu as pltpu
from jax.experimental.pallas import tpu_sc as plsc

NC = 2
NS = 16
NW = NC * NS
LANES = 16


def _deg_kernel(to2d, ones2d, zseg, *, n_pad, rows, seg):
    mesh = plsc.VectorSubcoreMesh(core_axis_name="c", subcore_axis_name="s")

    @functools.partial(
        pl.kernel,
        mesh=mesh,
        out_type=jax.ShapeDtypeStruct((NC, n_pad), jnp.float32),
        scratch_types=[
            pltpu.VMEM((rows, 128), jnp.int32),
            pltpu.VMEM((rows, 128), jnp.float32),
            pltpu.VMEM((seg,), jnp.float32),
            pltpu.VMEM_SHARED((n_pad,), jnp.float32),
        ],
    )
    def k(to_hbm, ones_hbm, z_hbm, out_hbm, idx_v, ones_v, seg_v, deg_sh):
        cid = lax.axis_index("c")
        sid = lax.axis_index("s")
        wid = cid * NS + sid
        pltpu.sync_copy(z_hbm, seg_v)
        pltpu.sync_copy(seg_v, deg_sh.at[pl.ds(sid * seg, seg)])
        pltpu.sync_copy(to_hbm.at[wid], idx_v)
        pltpu.sync_copy(ones_hbm, ones_v)
        plsc.subcore_barrier()
        pltpu.sync_copy(ones_v, deg_sh.at[idx_v], add=True)
        plsc.subcore_barrier()
        pltpu.sync_copy(deg_sh.at[pl.ds(sid * seg, seg)], seg_v)
        pltpu.sync_copy(seg_v, out_hbm.at[cid, pl.ds(sid * seg, seg)])

    return k(to2d, ones2d, zseg)


def _tables_kernel(pdeg3, x3, *, nrows):
    def body(pd_ref, x_ref, st_ref):
        deg = pd_ref[0] + pd_ref[1]
        rowsum = jnp.sum(x_ref[...], axis=-1)
        t = jnp.where(deg == 0.0, 0.0, lax.rsqrt(jnp.maximum(deg, 1.0)))
        st_ref[0] = t * rowsum
        st_ref[1] = t

    return pl.pallas_call(
        body,
        out_shape=jax.ShapeDtypeStruct((2, nrows, 128), jnp.float32),
    )(pdeg3, x3)


def _edge_kernel(st, from1, to1, *, n_pad, cpw):
    mesh = plsc.VectorSubcoreMesh(core_axis_name="c", subcore_axis_name="s")

    @functools.partial(
        pl.kernel,
        mesh=mesh,
        out_type=jax.ShapeDtypeStruct((NW, cpw), jnp.float32),
        scratch_types=[
            pltpu.VMEM((n_pad,), jnp.float32),
            pltpu.VMEM((n_pad,), jnp.float32),
            pltpu.VMEM((cpw,), jnp.int32),
            pltpu.VMEM((cpw,), jnp.int32),
            pltpu.VMEM((cpw,), jnp.float32),
        ],
    )
    def k(st_hbm, f_hbm, t_hbm, out_hbm, s_v, tt_v, fi_v, ti_v, o_v):
        cid = lax.axis_index("c")
        sid = lax.axis_index("s")
        wid = cid * NS + sid
        pltpu.sync_copy(st_hbm.at[0], s_v)
        pltpu.sync_copy(st_hbm.at[1], tt_v)
        pltpu.sync_copy(f_hbm.at[wid], fi_v)
        pltpu.sync_copy(t_hbm.at[wid], ti_v)

        def body(i, _):
            sl = pl.ds(i * LANES, LANES)
            sv = plsc.load_gather(s_v, [fi_v[sl]])
            tv = plsc.load_gather(tt_v, [ti_v[sl]])
            o_v[sl] = sv * tv
            return 0

        lax.fori_loop(0, cpw // LANES, body, 0)
        pltpu.sync_copy(o_v, out_hbm.at[wid])

    return k(st, from1, to1)


def kernel(x, edge_index):
    n, d = x.shape
    e = edge_index.shape[1]

    cpw = -(-e // NW)
    cpw = -(-cpw // 128) * 128
    rows = cpw // 128
    epad = cpw * NW
    n_pad = -(-n // 128) * 128
    if n_pad == n and epad > e:
        n_pad += 128
    seg = n_pad // NS

    ei = edge_index.astype(jnp.int32)
    from_p = jnp.concatenate([ei[0], jnp.zeros((epad - e,), jnp.int32)])
    to_p = jnp.concatenate([ei[1], jnp.full((epad - e,), jnp.int32(n), jnp.int32)])

    pdeg = _deg_kernel(
        to_p.reshape(NW, rows, 128),
        jnp.ones((rows, 128), jnp.float32),
        jnp.zeros((seg,), jnp.float32),
        n_pad=n_pad, rows=rows, seg=seg,
    )

    x3 = jnp.concatenate([x, jnp.zeros((n_pad - n, d), x.dtype)]) \
        .reshape(n_pad // 128, 128, d)
    st = _tables_kernel(pdeg.reshape(NC, n_pad // 128, 128), x3,
                        nrows=n_pad // 128)

    out = _edge_kernel(
        st.reshape(2, n_pad),
        from_p.reshape(NW, cpw),
        to_p.reshape(NW, cpw),
        n_pad=n_pad, cpw=cpw,
    )
    return out.reshape(-1)[:e]


# speedup vs baseline: 78.1186x; 78.1186x over previous
import functools

import jax
import jax.numpy as jnp
from jax import lax
from jax.experimental import pallas as pl
from jax.experimental.pallas import tpu as pltpu
from jax.experimental.pallas import tpu_sc as plsc

NC = 2
NS = 16
NW = NC * NS
LANES = 16


def _deg_kernel(to2d, ones2d, zseg, *, n_pad, cpw, seg):
    mesh = plsc.VectorSubcoreMesh(core_axis_name="c", subcore_axis_name="s")

    @functools.partial(
        pl.kernel,
        mesh=mesh,
        out_type=jax.ShapeDtypeStruct((NC * n_pad,), jnp.float32),
        scratch_types=[
            pltpu.VMEM((cpw,), jnp.int32),
            pltpu.VMEM((cpw,), jnp.float32),
            pltpu.VMEM((seg,), jnp.float32),
            pltpu.VMEM_SHARED((n_pad,), jnp.float32),
        ],
    )
    def k(to_hbm, ones_hbm, z_hbm, out_hbm, idx_v, ones_v, seg_v, deg_sh):
        cid = lax.axis_index("c")
        sid = lax.axis_index("s")
        wid = cid * NS + sid
        ebase = pl.multiple_of(wid * cpw, 8)
        obase = pl.multiple_of(cid * n_pad + sid * seg, 8)
        pltpu.sync_copy(z_hbm, seg_v)
        pltpu.sync_copy(seg_v, deg_sh.at[pl.ds(sid * seg, seg)])
        pltpu.sync_copy(to_hbm.at[pl.ds(ebase, cpw)], idx_v)
        pltpu.sync_copy(ones_hbm, ones_v)
        plsc.subcore_barrier()
        pltpu.sync_copy(ones_v, deg_sh.at[idx_v], add=True)
        plsc.subcore_barrier()
        pltpu.sync_copy(deg_sh.at[pl.ds(sid * seg, seg)], seg_v)
        pltpu.sync_copy(seg_v, out_hbm.at[pl.ds(obase, seg)])

    return k(to2d, ones2d, zseg)


def _tables_kernel(pdeg3, x3, *, nrows):
    def body(pd_ref, x_ref, st_ref):
        deg = pd_ref[0] + pd_ref[1]
        rowsum = jnp.sum(x_ref[...], axis=-1)
        t = jnp.where(deg == 0.0, 0.0, lax.rsqrt(jnp.maximum(deg, 1.0)))
        st_ref[0] = t * rowsum
        st_ref[1] = t

    return pl.pallas_call(
        body,
        out_shape=jax.ShapeDtypeStruct((2, nrows, 128), jnp.float32),
    )(pdeg3, x3)


def _edge_kernel(st, from1, to1, *, n_pad, cpw):
    mesh = plsc.VectorSubcoreMesh(core_axis_name="c", subcore_axis_name="s")

    @functools.partial(
        pl.kernel,
        mesh=mesh,
        out_type=jax.ShapeDtypeStruct((NW * cpw,), jnp.float32),
        scratch_types=[
            pltpu.VMEM((n_pad,), jnp.float32),
            pltpu.VMEM((n_pad,), jnp.float32),
            pltpu.VMEM((cpw,), jnp.int32),
            pltpu.VMEM((cpw,), jnp.int32),
            pltpu.VMEM((cpw,), jnp.float32),
        ],
        compiler_params=pltpu.CompilerParams(needs_layout_passes=False),
    )
    def k(st_hbm, f_hbm, t_hbm, out_hbm, s_v, tt_v, fi_v, ti_v, o_v):
        cid = lax.axis_index("c")
        sid = lax.axis_index("s")
        wid = cid * NS + sid
        ebase = pl.multiple_of(wid * cpw, 8)
        pltpu.sync_copy(st_hbm.at[pl.ds(0, n_pad)], s_v)
        pltpu.sync_copy(st_hbm.at[pl.ds(n_pad, n_pad)], tt_v)
        pltpu.sync_copy(f_hbm.at[pl.ds(ebase, cpw)], fi_v)
        pltpu.sync_copy(t_hbm.at[pl.ds(ebase, cpw)], ti_v)

        def body(i, _):
            sl = pl.ds(i * LANES, LANES)
            sv = plsc.load_gather(s_v, [fi_v[sl]])
            tv = plsc.load_gather(tt_v, [ti_v[sl]])
            o_v[sl] = sv * tv
            return 0

        lax.fori_loop(0, cpw // LANES, body, 0)
        pltpu.sync_copy(o_v, out_hbm.at[pl.ds(ebase, cpw)])

    return k(st, from1, to1)


def kernel(x, edge_index):
    n, d = x.shape
    e = edge_index.shape[1]

    cpw = -(-e // NW)
    cpw = -(-cpw // 128) * 128
    rows = cpw // 128
    epad = cpw * NW
    n_pad = -(-n // 128) * 128
    if n_pad == n and epad > e:
        n_pad += 128
    seg = n_pad // NS

    ei = edge_index.astype(jnp.int32)
    from_p = jnp.concatenate([ei[0], jnp.zeros((epad - e,), jnp.int32)])
    to_p = jnp.concatenate([ei[1], jnp.full((epad - e,), jnp.int32(n), jnp.int32)])

    pdeg = _deg_kernel(
        to_p,
        jnp.ones((cpw,), jnp.float32),
        jnp.zeros((seg,), jnp.float32),
        n_pad=n_pad, cpw=cpw, seg=seg,
    )

    x3 = jnp.concatenate([x, jnp.zeros((n_pad - n, d), x.dtype)]) \
        .reshape(n_pad // 128, 128, d)
    st = _tables_kernel(pdeg.reshape(NC, n_pad // 128, 128), x3,
                        nrows=n_pad // 128)

    out = _edge_kernel(
        st.reshape(2 * n_pad),
        from_p,
        to_p,
        n_pad=n_pad, cpw=cpw,
    )
    return out[:e]


# speedup vs baseline: 111.9205x; 1.4327x over previous
import functools

import jax
import jax.numpy as jnp
from jax import lax
from jax.experimental import pallas as pl
from jax.experimental.pallas import tpu as pltpu
from jax.experimental.pallas import tpu_sc as plsc

NC = 2
NS = 16
NW = NC * NS
LANES = 16


def _rowsum_kernel(x):
    def body(x_ref, out_ref):
        out_ref[...] = jnp.sum(x_ref[...], axis=-1)

    n = x.shape[0]
    return pl.pallas_call(
        body,
        out_shape=jax.ShapeDtypeStruct((n,), jnp.float32),
    )(x)


def _fused_sc_kernel(ei_flat, rs_pad, *, e, n_pad, cpw, seg):
    mesh = plsc.VectorSubcoreMesh(core_axis_name="c", subcore_axis_name="s")

    @functools.partial(
        pl.kernel,
        mesh=mesh,
        out_type=jax.ShapeDtypeStruct((e,), jnp.float32),
        scratch_types=[
            pltpu.VMEM((2 * cpw,), jnp.int32),
            pltpu.VMEM((2 * cpw,), jnp.float32),
            pltpu.VMEM((seg,), jnp.float32),
            pltpu.VMEM((seg,), jnp.float32),
            pltpu.VMEM((n_pad,), jnp.float32),
            pltpu.VMEM((n_pad,), jnp.float32),
            pltpu.VMEM((cpw,), jnp.int32),
            pltpu.VMEM((cpw,), jnp.int32),
            pltpu.VMEM((cpw,), jnp.float32),
            pltpu.VMEM_SHARED((n_pad,), jnp.float32),
            pltpu.VMEM_SHARED((n_pad,), jnp.float32),
            pltpu.VMEM_SHARED((n_pad,), jnp.float32),
        ],
        compiler_params=pltpu.CompilerParams(needs_layout_passes=False),
    )
    def k(ei_hbm, rs_hbm, out_hbm, idx_v, ones_v, seg_v, rs_v, s_v, tt_v,
          fi_v, ti_v, o_v, deg_sh, s_sh, t_sh):
        cid = lax.axis_index("c")
        sid = lax.axis_index("s")
        wid = cid * NS + sid
        sbase = pl.multiple_of(sid * seg, 8)
        ebase = pl.multiple_of(wid * cpw, 8)
        hbase = pl.multiple_of(e + sid * (2 * cpw), 8)

        ones16 = jnp.full((LANES,), 1.0, jnp.float32)
        zeros16 = jnp.zeros((LANES,), jnp.float32)

        def fill_ones(i, _):
            ones_v[pl.ds(i * LANES, LANES)] = ones16
            return 0

        lax.fori_loop(0, (2 * cpw) // LANES, fill_ones, 0, unroll=8)

        def fill_zero(i, _):
            seg_v[pl.ds(i * LANES, LANES)] = zeros16
            return 0

        lax.fori_loop(0, seg // LANES, fill_zero, 0, unroll=8)
        pltpu.sync_copy(seg_v, deg_sh.at[pl.ds(sbase, seg)])

        pltpu.sync_copy(ei_hbm.at[pl.ds(hbase, 2 * cpw)], idx_v)
        plsc.subcore_barrier()

        pltpu.sync_copy(ones_v, deg_sh.at[idx_v], add=True)
        plsc.subcore_barrier()

        pltpu.sync_copy(deg_sh.at[pl.ds(sbase, seg)], seg_v)
        pltpu.sync_copy(rs_hbm.at[pl.ds(sbase, seg)], rs_v)

        def table_step(i, _):
            sl = pl.ds(i * LANES, LANES)
            d = seg_v[sl]
            ibits = plsc.bitcast(d, jnp.int32)
            y = plsc.bitcast(
                jnp.full((LANES,), 0x5F3759DF, jnp.int32)
                - lax.shift_right_logical(ibits, 1),
                jnp.float32,
            )
            hd = 0.5 * d
            y = y * (1.5 - hd * y * y)
            y = y * (1.5 - hd * y * y)
            y = y * (1.5 - hd * y * y)
            t = jnp.where(d == 0.0, 0.0, y)
            seg_v[sl] = t
            rs_v[sl] = t * rs_v[sl]
            return 0

        lax.fori_loop(0, seg // LANES, table_step, 0, unroll=4)
        pltpu.sync_copy(rs_v, s_sh.at[pl.ds(sbase, seg)])
        pltpu.sync_copy(seg_v, t_sh.at[pl.ds(sbase, seg)])
        pltpu.sync_copy(ei_hbm.at[pl.ds(ebase, cpw)], fi_v)
        pltpu.sync_copy(ei_hbm.at[pl.ds(e + ebase, cpw)], ti_v)
        plsc.subcore_barrier()

        pltpu.sync_copy(s_sh, s_v)
        pltpu.sync_copy(t_sh, tt_v)

        def gather_step(i, _):
            sl = pl.ds(i * LANES, LANES)
            sv = plsc.load_gather(s_v, [fi_v[sl]])
            tv = plsc.load_gather(tt_v, [ti_v[sl]])
            o_v[sl] = sv * tv
            return 0

        lax.fori_loop(0, cpw // LANES, gather_step, 0, unroll=8)
        pltpu.sync_copy(o_v, out_hbm.at[pl.ds(ebase, cpw)])

    return k(ei_flat, rs_pad)


def kernel(x, edge_index):
    n, d = x.shape
    e = edge_index.shape[1]
    assert e % (NW * LANES) == 0, "edge count must split across subcores"
    cpw = e // NW
    n_pad = -(-n // (NS * LANES)) * (NS * LANES)
    seg = n_pad // NS

    ei_flat = edge_index.astype(jnp.int32).reshape(2 * e)
    rs = _rowsum_kernel(x)
    rs_pad = jnp.concatenate([rs, jnp.zeros((n_pad - n,), jnp.float32)])
    return _fused_sc_kernel(ei_flat, rs_pad, e=e, n_pad=n_pad, cpw=cpw, seg=seg)


# speedup vs baseline: 135.3776x; 1.2096x over previous
import functools

import jax
import jax.numpy as jnp
from jax import lax
from jax.experimental import pallas as pl
from jax.experimental.pallas import tpu as pltpu
from jax.experimental.pallas import tpu_sc as plsc

NC = 2
NS = 16
NW = NC * NS
LANES = 16


def _rowsum_kernel(x):
    def body(x_ref, out_ref):
        out_ref[...] = jnp.sum(x_ref[...], axis=-1)

    n = x.shape[0]
    return pl.pallas_call(
        body,
        out_shape=jax.ShapeDtypeStruct((n,), jnp.float32),
    )(x)


def _fused_sc_kernel(ei_flat, rs_pad, *, e, n_pad, cpw, seg):
    mesh = plsc.VectorSubcoreMesh(core_axis_name="c", subcore_axis_name="s")

    @functools.partial(
        pl.kernel,
        mesh=mesh,
        out_type=jax.ShapeDtypeStruct((e,), jnp.float32),
        scratch_types=[
            pltpu.VMEM((2 * cpw,), jnp.int32),
            pltpu.VMEM((2 * cpw,), jnp.float32),
            pltpu.VMEM((seg,), jnp.float32),
            pltpu.VMEM((seg,), jnp.float32),
            pltpu.VMEM((n_pad,), jnp.float32),
            pltpu.VMEM((n_pad,), jnp.float32),
            pltpu.VMEM((cpw,), jnp.int32),
            pltpu.VMEM((cpw,), jnp.int32),
            pltpu.VMEM((cpw,), jnp.float32),
            pltpu.VMEM_SHARED((n_pad,), jnp.float32),
            pltpu.VMEM_SHARED((n_pad,), jnp.float32),
            pltpu.VMEM_SHARED((n_pad,), jnp.float32),
            pltpu.SemaphoreType.DMA,
            pltpu.SemaphoreType.DMA,
            pltpu.SemaphoreType.DMA,
            pltpu.SemaphoreType.DMA,
        ],
        compiler_params=pltpu.CompilerParams(needs_layout_passes=False),
    )
    def k(ei_hbm, rs_hbm, out_hbm, idx_v, ones_v, seg_v, rs_v, s_v, tt_v,
          fi_v, ti_v, o_v, deg_sh, s_sh, t_sh, sem_h, sem_f, sem_t, sem_r):
        cid = lax.axis_index("c")
        sid = lax.axis_index("s")
        wid = cid * NS + sid
        sbase = pl.multiple_of(sid * seg, 8)
        ebase = pl.multiple_of(wid * cpw, 8)
        hbase = pl.multiple_of(e + sid * (2 * cpw), 8)

        hist_cp = pltpu.async_copy(ei_hbm.at[pl.ds(hbase, 2 * cpw)], idx_v,
                                   sem_h)
        from_cp = pltpu.async_copy(ei_hbm.at[pl.ds(ebase, cpw)], fi_v, sem_f)
        to_cp = pltpu.async_copy(ei_hbm.at[pl.ds(e + ebase, cpw)], ti_v,
                                 sem_t)
        rs_cp = pltpu.async_copy(rs_hbm.at[pl.ds(sbase, seg)], rs_v, sem_r)

        ones16 = jnp.full((LANES,), 1.0, jnp.float32)
        zeros16 = jnp.zeros((LANES,), jnp.float32)

        @plsc.parallel_loop(0, seg // LANES, unroll=8)
        def _(i):
            seg_v[pl.ds(i * LANES, LANES)] = zeros16

        pltpu.sync_copy(seg_v, deg_sh.at[pl.ds(sbase, seg)])

        @plsc.parallel_loop(0, (2 * cpw) // LANES, unroll=8)
        def _(i):
            ones_v[pl.ds(i * LANES, LANES)] = ones16

        hist_cp.wait()
        plsc.subcore_barrier()

        pltpu.sync_copy(ones_v, deg_sh.at[idx_v], add=True)
        plsc.subcore_barrier()

        pltpu.sync_copy(deg_sh.at[pl.ds(sbase, seg)], seg_v)
        rs_cp.wait()

        @plsc.parallel_loop(0, seg // LANES, unroll=4)
        def _(i):
            sl = pl.ds(i * LANES, LANES)
            d = seg_v[sl]
            ibits = plsc.bitcast(d, jnp.int32)
            y = plsc.bitcast(
                jnp.full((LANES,), 0x5F3759DF, jnp.int32)
                - lax.shift_right_logical(ibits, 1),
                jnp.float32,
            )
            hd = 0.5 * d
            y = y * (1.5 - hd * y * y)
            y = y * (1.5 - hd * y * y)
            y = y * (1.5 - hd * y * y)
            t = jnp.where(d == 0.0, 0.0, y)
            seg_v[sl] = t
            rs_v[sl] = t * rs_v[sl]

        pltpu.sync_copy(rs_v, s_sh.at[pl.ds(sbase, seg)])
        pltpu.sync_copy(seg_v, t_sh.at[pl.ds(sbase, seg)])
        plsc.subcore_barrier()

        pltpu.sync_copy(s_sh, s_v)
        pltpu.sync_copy(t_sh, tt_v)
        from_cp.wait()
        to_cp.wait()

        @plsc.parallel_loop(0, cpw // LANES, unroll=8)
        def _(i):
            sl = pl.ds(i * LANES, LANES)
            sv = plsc.load_gather(s_v, [fi_v[sl]])
            tv = plsc.load_gather(tt_v, [ti_v[sl]])
            o_v[sl] = sv * tv

        pltpu.sync_copy(o_v, out_hbm.at[pl.ds(ebase, cpw)])

    return k(ei_flat, rs_pad)


def kernel(x, edge_index):
    n, d = x.shape
    e = edge_index.shape[1]
    assert e % (NW * LANES) == 0, "edge count must split across subcores"
    cpw = e // NW
    n_pad = -(-n // (NS * LANES)) * (NS * LANES)
    seg = n_pad // NS

    ei_flat = edge_index.astype(jnp.int32).reshape(2 * e)
    rs = _rowsum_kernel(x)
    rs_pad = jnp.concatenate([rs, jnp.zeros((n_pad - n,), jnp.float32)])
    return _fused_sc_kernel(ei_flat, rs_pad, e=e, n_pad=n_pad, cpw=cpw, seg=seg)


# speedup vs baseline: 137.5455x; 1.0160x over previous
import functools

import jax
import jax.numpy as jnp
from jax import lax
from jax.experimental import pallas as pl
from jax.experimental.pallas import tpu as pltpu
from jax.experimental.pallas import tpu_sc as plsc

NC = 2
NS = 16
NW = NC * NS
LANES = 16


def _rowsum_kernel(x, n_pad):
    def body(x_ref, out_ref):
        out_ref[...] = jnp.sum(x_ref[...], axis=-1)

    d = x.shape[1]
    blk = 2048
    assert n_pad % blk == 0 and n_pad - blk < x.shape[0], \
        "every block must overlap valid rows"
    return pl.pallas_call(
        body,
        grid=(n_pad // blk,),
        in_specs=[pl.BlockSpec((blk, d), lambda i: (i, 0))],
        out_specs=pl.BlockSpec((blk,), lambda i: (i,)),
        out_shape=jax.ShapeDtypeStruct((n_pad,), jnp.float32),
    )(x)


def _fused_sc_kernel(ei_flat, rs_pad, *, e, n_pad, cpw, seg):
    mesh = plsc.VectorSubcoreMesh(core_axis_name="c", subcore_axis_name="s")

    @functools.partial(
        pl.kernel,
        mesh=mesh,
        out_type=jax.ShapeDtypeStruct((e,), jnp.float32),
        scratch_types=[
            pltpu.VMEM((2 * cpw,), jnp.int32),
            pltpu.VMEM((2 * cpw,), jnp.float32),
            pltpu.VMEM((seg,), jnp.float32),
            pltpu.VMEM((seg,), jnp.float32),
            pltpu.VMEM((n_pad,), jnp.float32),
            pltpu.VMEM((n_pad,), jnp.float32),
            pltpu.VMEM((cpw,), jnp.int32),
            pltpu.VMEM((cpw,), jnp.int32),
            pltpu.VMEM((cpw,), jnp.float32),
            pltpu.VMEM_SHARED((n_pad,), jnp.float32),
            pltpu.VMEM_SHARED((n_pad,), jnp.float32),
            pltpu.VMEM_SHARED((n_pad,), jnp.float32),
            pltpu.SemaphoreType.DMA,
            pltpu.SemaphoreType.DMA,
            pltpu.SemaphoreType.DMA,
            pltpu.SemaphoreType.DMA,
        ],
        compiler_params=pltpu.CompilerParams(needs_layout_passes=False),
    )
    def k(ei_hbm, rs_hbm, out_hbm, idx_v, ones_v, seg_v, rs_v, s_v, tt_v,
          fi_v, ti_v, o_v, deg_sh, s_sh, t_sh, sem_h, sem_f, sem_t, sem_r):
        cid = lax.axis_index("c")
        sid = lax.axis_index("s")
        wid = cid * NS + sid
        sbase = pl.multiple_of(sid * seg, 8)
        ebase = pl.multiple_of(wid * cpw, 8)
        hbase = pl.multiple_of(e + sid * (2 * cpw), 8)

        hist_cp = pltpu.async_copy(ei_hbm.at[pl.ds(hbase, 2 * cpw)], idx_v,
                                   sem_h)
        from_cp = pltpu.async_copy(ei_hbm.at[pl.ds(ebase, cpw)], fi_v, sem_f)
        to_cp = pltpu.async_copy(ei_hbm.at[pl.ds(e + ebase, cpw)], ti_v,
                                 sem_t)
        rs_cp = pltpu.async_copy(rs_hbm.at[pl.ds(sbase, seg)], rs_v, sem_r)

        ones16 = jnp.full((LANES,), 1.0, jnp.float32)
        zeros16 = jnp.zeros((LANES,), jnp.float32)

        @plsc.parallel_loop(0, seg // LANES, unroll=8)
        def _(i):
            seg_v[pl.ds(i * LANES, LANES)] = zeros16

        pltpu.sync_copy(seg_v, deg_sh.at[pl.ds(sbase, seg)])

        @plsc.parallel_loop(0, (2 * cpw) // LANES, unroll=8)
        def _(i):
            ones_v[pl.ds(i * LANES, LANES)] = ones16

        hist_cp.wait()
        plsc.subcore_barrier()

        pltpu.sync_copy(ones_v, deg_sh.at[idx_v], add=True)
        plsc.subcore_barrier()

        pltpu.sync_copy(deg_sh.at[pl.ds(sbase, seg)], seg_v)
        rs_cp.wait()

        @plsc.parallel_loop(0, seg // LANES, unroll=4)
        def _(i):
            sl = pl.ds(i * LANES, LANES)
            d = seg_v[sl]
            ibits = plsc.bitcast(d, jnp.int32)
            y = plsc.bitcast(
                jnp.full((LANES,), 0x5F3759DF, jnp.int32)
                - lax.shift_right_logical(ibits, 1),
                jnp.float32,
            )
            hd = 0.5 * d
            y = y * (1.5 - hd * y * y)
            y = y * (1.5 - hd * y * y)
            y = y * (1.5 - hd * y * y)
            t = jnp.where(d == 0.0, 0.0, y)
            seg_v[sl] = t
            rs_v[sl] = t * rs_v[sl]

        pltpu.sync_copy(rs_v, s_sh.at[pl.ds(sbase, seg)])
        pltpu.sync_copy(seg_v, t_sh.at[pl.ds(sbase, seg)])
        plsc.subcore_barrier()

        pltpu.sync_copy(s_sh, s_v)
        pltpu.sync_copy(t_sh, tt_v)
        from_cp.wait()
        to_cp.wait()

        @plsc.parallel_loop(0, cpw // LANES, unroll=8)
        def _(i):
            sl = pl.ds(i * LANES, LANES)
            sv = plsc.load_gather(s_v, [fi_v[sl]])
            tv = plsc.load_gather(tt_v, [ti_v[sl]])
            o_v[sl] = sv * tv

        pltpu.sync_copy(o_v, out_hbm.at[pl.ds(ebase, cpw)])

    return k(ei_flat, rs_pad)


def kernel(x, edge_index):
    n, d = x.shape
    e = edge_index.shape[1]
    assert e % (NW * LANES) == 0, "edge count must split across subcores"
    cpw = e // NW
    n_pad = -(-n // (NS * LANES)) * (NS * LANES)
    seg = n_pad // NS

    ei_flat = edge_index.astype(jnp.int32).reshape(2 * e)
    rs_pad = _rowsum_kernel(x, n_pad)
    return _fused_sc_kernel(ei_flat, rs_pad, e=e, n_pad=n_pad, cpw=cpw, seg=seg)
